# Initial kernel scaffold; baseline (speedup 1.0000x reference)
import jax
import jax.numpy as jnp
from jax.experimental import pallas as pl


def kernel(x, table, W0, b0, W1, b1, W2):
    raise NotImplementedError("write your pallas kernel here")


# speedup vs baseline: 5.5178x; 5.5178x over previous
import functools

import jax
import jax.numpy as jnp
from jax import lax
from jax.experimental import pallas as pl
from jax.experimental.pallas import tpu as pltpu
from jax.experimental.pallas import tpu_sc as plsc

NUM_SLOTS = 26
D = 16
HIST = 50
LANES = 16
NC = 2
NS = 16
NW = NC * NS
K = 64
IDS = K * HIST
JCOLS = 128
JROWS = IDS // JCOLS
ACC = K * NUM_SLOTS


def _segment_sums(table, gidx, roff, batch):
    chunks = batch // (NW * K)
    mesh = plsc.VectorSubcoreMesh(core_axis_name="c", subcore_axis_name="s")

    @functools.partial(
        pl.kernel,
        mesh=mesh,
        compiler_params=pltpu.CompilerParams(use_tc_tiling_on_sc=False),
        out_type=jax.ShapeDtypeStruct((batch * NUM_SLOTS, D), jnp.float32),
        scratch_types=[
            pltpu.VMEM((JROWS, JCOLS), jnp.int32),
            pltpu.VMEM((JROWS, JCOLS), jnp.int32),
            pltpu.VMEM((JROWS, JCOLS), jnp.int32),
            pltpu.VMEM((IDS, D), jnp.float32),
            pltpu.VMEM((ACC, D), jnp.float32),
            pltpu.VMEM_SHARED((NS * ACC, D), jnp.float32),
        ],
    )
    def sums_kernel(table_h, gidx_h, roff_h, out_h, gix_v, dix_v, roff_v,
                    buf_v, zero_v, acc_sh):
        cid = lax.axis_index("c")
        sid = lax.axis_index("s")
        wid = sid * NC + cid
        accbase = sid * ACC
        pltpu.sync_copy(roff_h, roff_v)

        def zero_body(i, _):
            zero_v[i, :] = jnp.zeros((D,), jnp.float32)
            return _

        lax.fori_loop(0, ACC, zero_body, None)

        def chunk_body(c, _):
            pltpu.sync_copy(gidx_h.at[wid, c], gix_v)
            def idx_body(t, _):
                r = t // (JCOLS // LANES)
                col = (t % (JCOLS // LANES)) * LANES
                g = gix_v[r, pl.ds(col, LANES)]
                dst = accbase + (roff_v[r, pl.ds(col, LANES)] + g % NUM_SLOTS)
                dix_v[r, pl.ds(col, LANES)] = dst
                return _

            lax.fori_loop(0, IDS // LANES, idx_body, None)
            pltpu.sync_copy(zero_v, acc_sh.at[pl.ds(accbase, ACC)])

            def gs_body(j, _):
                rows = buf_v.at[pl.ds(j * JCOLS, JCOLS)]
                pltpu.sync_copy(table_h.at[gix_v.at[j]], rows)
                pltpu.sync_copy(rows, acc_sh.at[dix_v.at[j]], add=True)
                return _

            lax.fori_loop(0, JROWS, gs_body, None)
            outbase = (wid * chunks + c) * ACC
            pltpu.sync_copy(acc_sh.at[pl.ds(accbase, ACC)],
                            out_h.at[pl.ds(outbase, ACC)])
            return _

        lax.fori_loop(0, chunks, chunk_body, None)

    return sums_kernel(table, gidx, roff)


def _mlp(sums, xi, expand, w0, b0, w1, b1, w2t, batch):
    blk = 1024
    fdim = NUM_SLOTS * D

    def body(sums_ref, x_ref, e_ref, w0_ref, b0_ref, w1_ref, b1_ref,
             w2_ref, o_ref):
        slots = x_ref[...] % NUM_SLOTS
        sidx = lax.broadcasted_iota(jnp.int32, (blk, NUM_SLOTS), 1)
        counts = jnp.zeros((blk, NUM_SLOTS), jnp.float32)
        for l in range(HIST):
            counts += (slots[:, l:l + 1] == sidx).astype(jnp.float32)
        recip = 1.0 / jnp.maximum(counts, 1.0)
        scale = jnp.dot(recip, e_ref[...], preferred_element_type=jnp.float32)
        feat = sums_ref[...] * scale
        h = jnp.dot(feat, w0_ref[...], preferred_element_type=jnp.float32)
        h = jnp.maximum(h + b0_ref[...], 0.0)
        h = jnp.dot(h, w1_ref[...], preferred_element_type=jnp.float32)
        h = jnp.maximum(h + b1_ref[...], 0.0)
        z = jnp.sum(h * w2_ref[...], axis=1, keepdims=True)
        o_ref[...] = 1.0 / (1.0 + jnp.exp(-z))

    full = lambda shape: pl.BlockSpec(shape, lambda i: (0, 0))
    return pl.pallas_call(
        body,
        grid=(batch // blk,),
        in_specs=[
            pl.BlockSpec((blk, fdim), lambda i: (i, 0)),
            pl.BlockSpec((blk, HIST), lambda i: (i, 0)),
            full((NUM_SLOTS, fdim)),
            full((fdim, 64)),
            full((1, 64)),
            full((64, 16)),
            full((1, 16)),
            full((1, 16)),
        ],
        out_specs=pl.BlockSpec((blk, 1), lambda i: (i, 0)),
        out_shape=jax.ShapeDtypeStruct((batch, 1), jnp.float32),
    )(sums, xi, expand, w0, b0, w1, b1, w2t)


def kernel(x, table, W0, b0, W1, b1, W2):
    batch, hist = x.shape
    xi = x.astype(jnp.int32)
    chunks = batch // (NW * K)
    gidx = xi.reshape(NW, chunks, JROWS, JCOLS)
    roff = ((jnp.arange(IDS, dtype=jnp.int32) // HIST)
            * NUM_SLOTS).reshape(JROWS, JCOLS)
    sums = _segment_sums(table, gidx, roff, batch)
    sums2d = sums.reshape(batch, NUM_SLOTS * D)
    expand = jnp.repeat(jnp.eye(NUM_SLOTS, dtype=jnp.float32), D, axis=1)
    return _mlp(sums2d, xi, expand, W0, b0.reshape(1, 64),
                W1, b1.reshape(1, 16), W2.reshape(1, 16), batch)
